# baseline (speedup 1.0000x reference)
import functools

import jax
import jax.numpy as jnp
from jax import lax
from jax.experimental import pallas as pl
from jax.experimental.pallas import tpu as pltpu, tpu_sc as plsc

N = 4096
BH = BW = 32
GRID = N // BH
NNZ = 1638
NW = 32
PER_W = 52
PER_W_PAD = 64
RB = 13
NROUNDS = PER_W // RB

_mesh = plsc.VectorSubcoreMesh(core_axis_name="c", subcore_axis_name="s")


@functools.partial(
    pl.kernel,
    out_type=jax.ShapeDtypeStruct((NNZ, BH, 4 * BW), jnp.float32),
    mesh=_mesh,
    scratch_types=[
        pltpu.VMEM((128,), jnp.int32),
        pltpu.VMEM((128,), jnp.int32),
        pltpu.VMEM((2 * RB, BH, 4 * BW), jnp.float32),
        pltpu.SemaphoreType.DMA,
        pltpu.SemaphoreType.DMA,
        pltpu.SemaphoreType.DMA,
    ],
)
def _gather_blocks(x, r2d, c2d, out, rows_v, cols_v, wide,
                   sem_in0, sem_in1, sem_out):
    wid = lax.axis_index("s") * 2 + lax.axis_index("c")
    pltpu.sync_copy(r2d.at[wid], rows_v)
    pltpu.sync_copy(c2d.at[wid], cols_v)

    blk = []
    for j in range((PER_W + 15) // 16):
        r16 = rows_v[pl.ds(j * 16, 16)]
        c16 = cols_v[pl.ds(j * 16, 16)]
        for k in range(16):
            t = j * 16 + k
            if t >= PER_W:
                break
            blk.append((wid * PER_W + t, r16[k], c16[k]))

    def in_copy(g, m):
        _, r, c = blk[g * RB + m]
        slot = (g % 2) * RB + m
        return pltpu.make_async_copy(
            x.at[pl.ds(r * BH, BH), pl.ds((c >> 2) * (4 * BW), 4 * BW)],
            wide.at[slot], sem_in1 if g % 2 else sem_in0)

    def out_copy(g, m):
        _, _, c = blk[g * RB + m]
        slot = (g % 2) * RB + m
        return pltpu.make_async_copy(
            wide.at[slot], out.at[blk[g * RB + m][0]], sem_out)

    for m in range(RB):
        in_copy(0, m).start()
    for g in range(NROUNDS):
        if g + 1 < NROUNDS:
            if g >= 1:
                for m in range(RB):
                    n = blk[(g - 1) * RB + m][0]
                    @pl.when(n < NNZ)
                    def _():
                        out_copy(g - 1, m).wait()
            for m in range(RB):
                in_copy(g + 1, m).start()
        for m in range(RB):
            in_copy(g, m).wait()
        for m in range(RB):
            n = blk[g * RB + m][0]
            @pl.when(n < NNZ)
            def _():
                out_copy(g, m).start()
    for g in (NROUNDS - 2, NROUNDS - 1):
        for m in range(RB):
            n = blk[g * RB + m][0]
            @pl.when(n < NNZ)
            def _():
                out_copy(g, m).wait()


def kernel(x, active_indices):
    ai = active_indices.astype(jnp.int32)
    pad = jnp.zeros((NW * PER_W, 2), jnp.int32).at[:NNZ].set(ai)
    r2d = jnp.zeros((NW, 128), jnp.int32).at[:, :PER_W].set(
        pad[:, 0].reshape(NW, PER_W))
    c2d = jnp.zeros((NW, 128), jnp.int32).at[:, :PER_W].set(
        pad[:, 1].reshape(NW, PER_W))
    wide_out = _gather_blocks(x, r2d, c2d)
    off = (ai[:, 1] % 4) * BW
    return jax.vmap(
        lambda w, o: jax.lax.dynamic_slice(w, (0, o), (BH, BW)))(wide_out, off)


# speedup vs baseline: 2.8053x; 2.8053x over previous
import functools

import jax
import jax.numpy as jnp
from jax import lax
from jax.experimental import pallas as pl
from jax.experimental.pallas import tpu as pltpu, tpu_sc as plsc

N = 4096
BH = BW = 32
GRID = N // BH
NNZ = 1638
NW = 32
PER_W = 52
RB = 13
NROUNDS = PER_W // RB
BLK = BH * BW

_mesh = plsc.VectorSubcoreMesh(core_axis_name="c", subcore_axis_name="s")


@functools.partial(
    pl.kernel,
    out_type=jax.ShapeDtypeStruct((NNZ * BLK,), jnp.float32),
    mesh=_mesh,
    scratch_types=[
        pltpu.VMEM((128,), jnp.int32),
        pltpu.VMEM((128,), jnp.int32),
        pltpu.VMEM((2 * RB, BH, 4 * BW), jnp.float32),
        pltpu.VMEM((RB * BLK,), jnp.float32),
        pltpu.SemaphoreType.DMA,
        pltpu.SemaphoreType.DMA,
        pltpu.SemaphoreType.DMA,
    ],
)
def _gather_blocks(x, r2d, c2d, out, rows_v, cols_v, wide, stage,
                   sem_in0, sem_in1, sem_out):
    wid = lax.axis_index("s") * 2 + lax.axis_index("c")
    pltpu.sync_copy(r2d.at[wid], rows_v)
    pltpu.sync_copy(c2d.at[wid], cols_v)

    rc = []
    for j in range((PER_W + 15) // 16):
        r16 = rows_v[pl.ds(j * 16, 16)]
        c16 = cols_v[pl.ds(j * 16, 16)]
        for k in range(16):
            if j * 16 + k >= PER_W:
                break
            rc.append((r16[k], c16[k]))

    def in_copy(g, m):
        r, c = rc[g * RB + m]
        return pltpu.make_async_copy(
            x.at[pl.ds(r * BH, BH), pl.ds((c >> 2) * (4 * BW), 4 * BW)],
            wide.at[(g % 2) * RB + m], sem_in1 if g % 2 else sem_in0)

    def out_copy(g):
        return pltpu.make_async_copy(
            stage, out.at[pl.ds((wid * PER_W + g * RB) * BLK, RB * BLK)],
            sem_out)

    for m in range(RB):
        in_copy(0, m).start()
    for g in range(NROUNDS):
        if g + 1 < NROUNDS:
            for m in range(RB):
                in_copy(g + 1, m).start()
        for m in range(RB):
            in_copy(g, m).wait()

        offs = [(c & 3) * BW for _, c in rc[g * RB:(g + 1) * RB]]
        slot0 = (g % 2) * RB

        def extract_row(i, _):
            for m in range(RB):
                src = wide.at[slot0 + m]
                dst_base = m * BLK + i * BW
                for h in (0, 16):
                    stage[pl.ds(dst_base + h, 16)] = src[i, pl.ds(offs[m] + h, 16)]
            return _

        valid = (wid < NW - 1) if g >= 2 else None
        if valid is None:
            lax.fori_loop(0, BH, extract_row, 0, unroll=4)
            out_copy(g).start()
            out_copy(g).wait()
        else:
            @pl.when(valid)
            def _():
                lax.fori_loop(0, BH, extract_row, 0, unroll=4)
                out_copy(g).start()
                out_copy(g).wait()


def kernel(x, active_indices):
    ai = active_indices.astype(jnp.int32)
    pad = jnp.zeros((NW * PER_W, 2), jnp.int32).at[:NNZ].set(ai)
    r2d = jnp.zeros((NW, 128), jnp.int32).at[:, :PER_W].set(
        pad[:, 0].reshape(NW, PER_W))
    c2d = jnp.zeros((NW, 128), jnp.int32).at[:, :PER_W].set(
        pad[:, 1].reshape(NW, PER_W))
    flat = _gather_blocks(x, r2d, c2d)
    return flat.reshape(NNZ, BH, BW)
